# baseline (speedup 1.0000x reference)
import functools

import numpy as np
import jax
import jax.numpy as jnp
from jax import lax
from jax.experimental import pallas as pl
from jax.experimental.pallas import tpu as pltpu
from jax.experimental.pallas import tpu_sc as plsc

_D = 128
_L = 80
_K = 5
_NEG = 5
_SLOTS = 2 * _K
_NNEG = _L * _SLOTS * _NEG
_NW = 32
_B = 4096
_BPW = _B // _NW

_NPAIRS = float(sum(min(i + _K, _L - 1) - max(i - _K, 0) for i in range(_L)))


def _static_table():
    tab = np.zeros(_B, np.int32)
    for g in range(_B):
        n = g - _L
        if 0 <= n < _NNEG:
            o = n // (_SLOTS * _NEG)
            slot = (n % (_SLOTS * _NEG)) // _NEG
            w = min(o + _K, _L - 1) - max(o - _K, 0)
            tab[g] = o | (int(slot < w) << 16)
    return tab


_TAB_NP = _static_table()


def _softplus16(x):
    y = jnp.exp(-jnp.abs(x))
    t = y / (y + 2.0)
    t2 = t * t
    p = t2 * jnp.float32(1.0 / 9.0) + jnp.float32(1.0 / 7.0)
    p = p * t2 + jnp.float32(1.0 / 5.0)
    p = p * t2 + jnp.float32(1.0 / 3.0)
    p = p * t2 + jnp.float32(1.0)
    return jnp.maximum(x, 0.0) + 2.0 * t * p


def _batch_dots(a_ref, a_rows, b_ref, b_rows):
    def step(s, accs):
        d0 = jnp.zeros((16,), jnp.int32) + s * 16
        for t in range(16):
            dv = d0 + t
            ga = plsc.load_gather(a_ref, [a_rows, dv])
            gb = plsc.load_gather(b_ref, [b_rows, dv])
            accs = tuple(a + ga * gb if q == (t % 4) else a
                         for q, a in enumerate(accs))
        return accs

    z = jnp.zeros((16,), jnp.float32)
    a0, a1, a2, a3 = lax.fori_loop(0, _D // 16, step, (z, z, z, z))
    return a0 + a1 + a2 + a3


def _sc_loss(table, mp, neg, tab):
    mesh = plsc.VectorSubcoreMesh(core_axis_name="c", subcore_axis_name="s")

    @functools.partial(
        pl.kernel,
        mesh=mesh,
        out_type=jax.ShapeDtypeStruct((_NW, 16), jnp.float32),
        scratch_types=[
            pltpu.VMEM((_L,), jnp.int32),
            pltpu.VMEM((_BPW,), jnp.int32),
            pltpu.VMEM((_L, _D), jnp.float32),
            pltpu.VMEM((_BPW, _D), jnp.float32),
            pltpu.VMEM((_BPW,), jnp.int32),
            pltpu.VMEM((16,), jnp.float32),
            pltpu.SemaphoreType.DMA,
            pltpu.SemaphoreType.DMA,
        ],
        compiler_params=pltpu.CompilerParams(needs_layout_passes=False),
    )
    def body(table_hbm, mp_hbm, neg_hbm, tab_hbm, out_hbm,
             mp_v, idx_v, walk_v, rows_v, tab_v, acc_v, sem_w, sem_r):
        wid = lax.axis_index("s") * 2 + lax.axis_index("c")
        base = wid * _BPW

        pltpu.sync_copy(mp_hbm, mp_v)
        pltpu.sync_copy(tab_hbm.at[pl.ds(base, _BPW)], tab_v)

        @pl.when(wid == 0)
        def _():
            pltpu.sync_copy(mp_hbm, idx_v.at[pl.ds(0, _L)])
            pltpu.sync_copy(neg_hbm.at[pl.ds(0, _BPW - _L)],
                            idx_v.at[pl.ds(_L, _BPW - _L)])

        @pl.when((wid > 0) & (wid < _NW - 1))
        def _():
            pltpu.sync_copy(neg_hbm.at[pl.ds(base - _L, _BPW)], idx_v)

        @pl.when(wid == _NW - 1)
        def _():
            tail = _NNEG - ((_NW - 1) * _BPW - _L)
            pltpu.sync_copy(neg_hbm.at[pl.ds(_NNEG - tail, tail)],
                            idx_v.at[pl.ds(0, tail)])
            for t in range(tail, _BPW, 16):
                idx_v[pl.ds(t, 16)] = jnp.zeros((16,), jnp.int32)

        cw = pltpu.async_copy(table_hbm.at[mp_v], walk_v, sem_w)
        cr = pltpu.async_copy(table_hbm.at[idx_v], rows_v, sem_r)
        cw.wait()

        total = jnp.zeros((16,), jnp.float32)

        for b in range(2):
            q = lax.iota(jnp.int32, 16) + 16 * b
            p = ((q >= _SLOTS).astype(jnp.int32)
                 + (q >= 2 * _SLOTS).astype(jnp.int32))
            k = q - p * _SLOTS
            off = k - _K + (k >= _K).astype(jnp.int32)
            widv = jnp.zeros((16,), jnp.int32) + wid
            iv = widv + _NW * p
            jv = iv + off
            okv = ((iv < _L) & (jv >= 0) & (jv < _L)
                   & (q < 3 * _SLOTS)).astype(jnp.float32)
            ia = jnp.minimum(iv, _L - 1)
            ja = jnp.clip(jv, 0, _L - 1)
            d = _batch_dots(walk_v, ia, walk_v, ja)
            total = total + _softplus16(okv * (30.0 - d) - 30.0)

        cr.wait()

        for b in range(_BPW // 16):
            rows = lax.iota(jnp.int32, 16) + 16 * b
            tv = tab_v[pl.ds(16 * b, 16)]
            ov = tv & jnp.int32(0xFFFF)
            mv = (tv >> 16).astype(jnp.float32)
            d = _batch_dots(rows_v, rows, walk_v, ov)
            total = total + _softplus16(mv * (d + 30.0) - 30.0)

        acc_v[...] = total
        pltpu.sync_copy(acc_v, out_hbm.at[wid])

    return body(table, mp, neg, tab)


def kernel(MP, neg_samples, X):
    mp = MP.astype(jnp.int32)
    neg = neg_samples.astype(jnp.int32).reshape(-1)
    tab = jnp.asarray(_TAB_NP)
    partials = _sc_loss(X, mp, neg, tab)
    return jnp.sum(partials) / jnp.float32(_NPAIRS)


# speedup vs baseline: 1.7286x; 1.7286x over previous
import functools

import jax
import jax.numpy as jnp
from jax import lax
from jax.experimental import pallas as pl
from jax.experimental.pallas import tpu as pltpu
from jax.experimental.pallas import tpu_sc as plsc

_D = 128
_L = 80
_K = 5
_NEG = 5
_SLOTS = 2 * _K
_NNEG = _L * _SLOTS * _NEG
_NROWS = _L + _NNEG
_NW = 32
_B = 4096
_BPW = _B // _NW


def _sc_gather(table, mp, neg):
    mesh = plsc.VectorSubcoreMesh(core_axis_name="c", subcore_axis_name="s")

    @functools.partial(
        pl.kernel,
        mesh=mesh,
        out_type=jax.ShapeDtypeStruct((_B, _D), jnp.float32),
        scratch_types=[
            pltpu.VMEM((_BPW,), jnp.int32),
            pltpu.VMEM((_BPW, _D), jnp.float32),
            pltpu.SemaphoreType.DMA,
        ],
    )
    def gather_kernel(table_hbm, mp_hbm, neg_hbm, out_hbm, idx_v, rows_v, sem):
        wid = lax.axis_index("s") * 2 + lax.axis_index("c")
        base = wid * _BPW

        @pl.when(wid == 0)
        def _():
            pltpu.sync_copy(mp_hbm, idx_v.at[pl.ds(0, _L)])
            pltpu.sync_copy(neg_hbm.at[pl.ds(0, _BPW - _L)],
                            idx_v.at[pl.ds(_L, _BPW - _L)])

        @pl.when((wid > 0) & (wid < _NW - 1))
        def _():
            pltpu.sync_copy(neg_hbm.at[pl.ds(base - _L, _BPW)], idx_v)

        @pl.when(wid == _NW - 1)
        def _():
            tail = _NNEG - ((_NW - 1) * _BPW - _L)
            pltpu.sync_copy(neg_hbm.at[pl.ds(_NNEG - tail, tail)],
                            idx_v.at[pl.ds(0, tail)])
            for t in range(tail, _BPW, 16):
                idx_v[pl.ds(t, 16)] = jnp.zeros((16,), jnp.int32)

        pltpu.async_copy(table_hbm.at[idx_v], rows_v, sem).wait()
        pltpu.sync_copy(rows_v, out_hbm.at[pl.ds(base, _BPW)])

    return gather_kernel(table, mp, neg)


def _tc_loss_kernel(rows_ref, out_ref):
    walk = rows_ref[0:_L, :]
    negs = rows_ref[_L:_L + _NNEG, :]

    gram = lax.dot_general(walk, walk, (((1,), (1,)), ((), ())),
                           preferred_element_type=jnp.float32)
    ii = lax.broadcasted_iota(jnp.int32, (_L, _L), 0)
    jj = lax.broadcasted_iota(jnp.int32, (_L, _L), 1)
    dij = jnp.abs(ii - jj)
    mask_pos = (dij >= 1) & (dij <= _K)

    negs3 = negs.reshape(_L, _SLOTS * _NEG, _D)
    nd = jnp.sum(negs3 * walk[:, None, :], axis=-1)
    i2 = lax.broadcasted_iota(jnp.int32, (_L, _SLOTS * _NEG), 0)
    slot = lax.broadcasted_iota(jnp.int32, (_L, _SLOTS * _NEG), 1) // _NEG
    wsize = jnp.minimum(i2 + _K, _L - 1) - jnp.maximum(i2 - _K, 0)
    mask_neg = slot < wsize

    def softplus(x):
        return jnp.maximum(x, 0.0) + jnp.log1p(jnp.exp(-jnp.abs(x)))

    pos_loss = jnp.sum(jnp.where(mask_pos, softplus(-gram), 0.0))
    neg_loss = jnp.sum(jnp.where(mask_neg, softplus(nd), 0.0))
    n_pairs = jnp.sum(mask_pos.astype(jnp.float32))
    out_ref[0, 0] = (pos_loss + neg_loss) / n_pairs


def kernel(MP, neg_samples, X):
    mp = MP.astype(jnp.int32)
    neg = neg_samples.astype(jnp.int32).reshape(-1)
    rows = _sc_gather(X, mp, neg)
    loss = pl.pallas_call(
        _tc_loss_kernel,
        out_shape=jax.ShapeDtypeStruct((1, 1), jnp.float32),
        out_specs=pl.BlockSpec(memory_space=pltpu.SMEM),
    )(rows)
    return loss[0, 0]
